# baseline (speedup 1.0000x reference)
import functools

import jax
import jax.numpy as jnp
from jax import lax
from jax.experimental import pallas as pl
from jax.experimental.pallas import tpu as pltpu
from jax.experimental.pallas import tpu_sc as plsc

B = 16384
N_ROWS = 100000
WVD = 300
WVDP = 384
CAT = 2 * WVDP
LATENT = 2048
EMB = 512

NC = 2
NS = 16
NW = NC * NS
BPW = B // NW
CH = 128
NCH = BPW // CH
NSTEP = 2 * NCH

_sc_mesh = plsc.VectorSubcoreMesh(core_axis_name="c", subcore_axis_name="s")


@functools.partial(
    pl.kernel,
    mesh=_sc_mesh,
    out_type=jax.ShapeDtypeStruct((B, CAT), jnp.float32),
    scratch_types=[
        pltpu.VMEM((NSTEP, CH), jnp.int32),
        pltpu.VMEM((2, CH, WVDP), jnp.float32),
        pltpu.SemaphoreType.DMA,
        pltpu.SemaphoreType.DMA,
    ],
)
def _gather_sc(attrs_hbm, objs_hbm, attr_t, obj_t,
               cat, idx_v, rows, sem_g, sem_w):
    wid = lax.axis_index("s") * NC + lax.axis_index("c")
    base = wid * BPW

    steps = []
    for ti, (idxs_hbm, tbl) in enumerate(((attrs_hbm, attr_t),
                                          (objs_hbm, obj_t))):
        for ci in range(NCH):
            steps.append((ti * NCH + ci, idxs_hbm, ci * CH, tbl, ti * WVDP))

    idx_cps = [
        pltpu.async_copy(idxs_hbm.at[pl.ds(base + off, CH)], idx_v.at[k],
                         sem_g)
        for k, idxs_hbm, off, _, _ in steps
    ]
    for cp in idx_cps:
        cp.wait()

    def start_gather(k):
        _, _, _, tbl, _ = steps[k]
        return pltpu.async_copy(tbl.at[idx_v.at[k], pl.ds(0, WVDP)],
                                rows.at[k % 2], sem_g)

    def start_write(k):
        _, _, off, _, coff = steps[k]
        return pltpu.async_copy(
            rows.at[k % 2],
            cat.at[pl.ds(base + off, CH), pl.ds(coff, WVDP)], sem_w)

    gathers = {0: start_gather(0)}
    writes = {}
    for k in range(NSTEP):
        if k + 1 < NSTEP:
            if k - 1 in writes:
                writes.pop(k - 1).wait()
            gathers[k + 1] = start_gather(k + 1)
        gathers.pop(k).wait()
        writes[k] = start_write(k)
    for k in sorted(writes):
        writes.pop(k).wait()


RB = 512


def _mlp_body(e_ref, mask_ref, w1_ref, b1_ref, w2_ref, b2_ref, out_ref):
    e = jnp.where(mask_ref[...] > 0, e_ref[...], 0.0)
    h = jnp.dot(e, w1_ref[...], preferred_element_type=jnp.float32) + b1_ref[...]
    h = jnp.maximum(h, 0.01 * h)
    out = jnp.dot(h, w2_ref[...], preferred_element_type=jnp.float32) + b2_ref[...]
    ssq = jnp.sum(out * out, axis=1, keepdims=True)
    out_ref[...] = out / jnp.maximum(jnp.sqrt(ssq), 1e-12)


_mlp_tc = pl.pallas_call(
    _mlp_body,
    grid=(B // RB,),
    in_specs=[
        pl.BlockSpec((RB, CAT), lambda i: (i, 0)),
        pl.BlockSpec((1, CAT), lambda i: (0, 0)),
        pl.BlockSpec((CAT, LATENT), lambda i: (0, 0)),
        pl.BlockSpec((1, LATENT), lambda i: (0, 0)),
        pl.BlockSpec((LATENT, EMB), lambda i: (0, 0)),
        pl.BlockSpec((1, EMB), lambda i: (0, 0)),
    ],
    out_specs=pl.BlockSpec((RB, EMB), lambda i: (i, 0)),
    out_shape=jax.ShapeDtypeStruct((B, EMB), jnp.float32),
)


def kernel(attrs, objs, attr_table, obj_table, W1, b1, W2, b2):
    attrs = attrs.astype(jnp.int32)
    objs = objs.astype(jnp.int32)
    cat = _gather_sc(attrs, objs, attr_table, obj_table)
    w1 = jnp.zeros((CAT, LATENT), jnp.float32)
    w1 = w1.at[:WVD].set(W1[:WVD]).at[WVDP:WVDP + WVD].set(W1[WVD:])
    col = jnp.arange(CAT, dtype=jnp.int32)[None, :]
    mask = ((col % WVDP) < WVD).astype(jnp.float32)
    return _mlp_tc(cat, mask, w1,
                   b1.reshape(1, LATENT), W2, b2.reshape(1, EMB))


# speedup vs baseline: 1.0091x; 1.0091x over previous
import functools

import jax
import jax.numpy as jnp
from jax import lax
from jax.experimental import pallas as pl
from jax.experimental.pallas import tpu as pltpu
from jax.experimental.pallas import tpu_sc as plsc

B = 16384
N_ROWS = 100000
WVD = 300
WVDP = 384
CAT = 2 * WVDP
LATENT = 2048
EMB = 512

NC = 2
NS = 16
NW = NC * NS
BPW = B // NW
CH = 128
NCH = BPW // CH
NSTEP = 2 * NCH

_sc_mesh = plsc.VectorSubcoreMesh(core_axis_name="c", subcore_axis_name="s")


@functools.partial(
    pl.kernel,
    mesh=_sc_mesh,
    out_type=jax.ShapeDtypeStruct((B, CAT), jnp.float32),
    scratch_types=[
        pltpu.VMEM((NSTEP, CH), jnp.int32),
        pltpu.VMEM((2, CH, WVDP), jnp.float32),
        pltpu.SemaphoreType.DMA,
        pltpu.SemaphoreType.DMA,
    ],
)
def _gather_sc(attrs_hbm, objs_hbm, attr_t, obj_t,
               cat, idx_v, rows, sem_g, sem_w):
    wid = lax.axis_index("s") * NC + lax.axis_index("c")
    base = wid * BPW

    steps = []
    for ti, (idxs_hbm, tbl) in enumerate(((attrs_hbm, attr_t),
                                          (objs_hbm, obj_t))):
        for ci in range(NCH):
            steps.append((ti * NCH + ci, idxs_hbm, ci * CH, tbl, ti * WVDP))

    idx_cps = [
        pltpu.async_copy(idxs_hbm.at[pl.ds(base + off, CH)], idx_v.at[k],
                         sem_g)
        for k, idxs_hbm, off, _, _ in steps
    ]
    for cp in idx_cps:
        cp.wait()

    def start_gather(k):
        _, _, _, tbl, _ = steps[k]
        return pltpu.async_copy(tbl.at[idx_v.at[k], pl.ds(0, WVDP)],
                                rows.at[k % 2], sem_g)

    def start_write(k):
        _, _, off, _, coff = steps[k]
        return pltpu.async_copy(
            rows.at[k % 2],
            cat.at[pl.ds(base + off, CH), pl.ds(coff, WVDP)], sem_w)

    gathers = {0: start_gather(0)}
    writes = {}
    for k in range(NSTEP):
        if k + 1 < NSTEP:
            if k - 1 in writes:
                writes.pop(k - 1).wait()
            gathers[k + 1] = start_gather(k + 1)
        gathers.pop(k).wait()
        writes[k] = start_write(k)
    for k in sorted(writes):
        writes.pop(k).wait()


RB = 1024


def _mlp_body(e_ref, mask_ref, w1_ref, b1_ref, w2_ref, b2_ref, out_ref):
    e = jnp.where(mask_ref[...] > 0, e_ref[...], 0.0)
    h = jnp.dot(e, w1_ref[...], preferred_element_type=jnp.float32) + b1_ref[...]
    h = jnp.maximum(h, 0.01 * h)
    out = jnp.dot(h, w2_ref[...], preferred_element_type=jnp.float32) + b2_ref[...]
    ssq = jnp.sum(out * out, axis=1, keepdims=True)
    out_ref[...] = out / jnp.maximum(jnp.sqrt(ssq), 1e-12)


_mlp_tc = pl.pallas_call(
    _mlp_body,
    grid=(B // RB,),
    in_specs=[
        pl.BlockSpec((RB, CAT), lambda i: (i, 0)),
        pl.BlockSpec((1, CAT), lambda i: (0, 0)),
        pl.BlockSpec((CAT, LATENT), lambda i: (0, 0)),
        pl.BlockSpec((1, LATENT), lambda i: (0, 0)),
        pl.BlockSpec((LATENT, EMB), lambda i: (0, 0)),
        pl.BlockSpec((1, EMB), lambda i: (0, 0)),
    ],
    out_specs=pl.BlockSpec((RB, EMB), lambda i: (i, 0)),
    out_shape=jax.ShapeDtypeStruct((B, EMB), jnp.float32),
)


def kernel(attrs, objs, attr_table, obj_table, W1, b1, W2, b2):
    attrs = attrs.astype(jnp.int32)
    objs = objs.astype(jnp.int32)
    cat = _gather_sc(attrs, objs, attr_table, obj_table)
    w1 = jnp.zeros((CAT, LATENT), jnp.float32)
    w1 = w1.at[:WVD].set(W1[:WVD]).at[WVDP:WVDP + WVD].set(W1[WVD:])
    col = jnp.arange(CAT, dtype=jnp.int32)[None, :]
    mask = ((col % WVDP) < WVD).astype(jnp.float32)
    return _mlp_tc(cat, mask, w1,
                   b1.reshape(1, LATENT), W2, b2.reshape(1, EMB))
